# baseline (speedup 1.0000x reference)
import functools

import jax
import jax.numpy as jnp
from jax import lax
from jax.experimental import pallas as pl
from jax.experimental.pallas import tpu as pltpu
from jax.experimental.pallas import tpu_sc as plsc

_NC = 2
_NS = 16
_NW = _NC * _NS


def _sc_onehot_body(dur_hbm, oh_hbm, dur_v, oh_v, *, s_per_w, batches, words):
    wid = lax.axis_index("s") * _NC + lax.axis_index("c")
    slab = s_per_w * batches * 128
    pltpu.sync_copy(dur_hbm, dur_v)

    zeros16 = jnp.zeros((16,), jnp.float32)

    def zero_fill(i, carry):
        for u in range(16):
            oh_v[pl.ds(i * 256 + u * 16, 16)] = zeros16
        return carry

    lax.fori_loop(0, slab // 256, zero_fill, 0)

    s0 = wid * s_per_w
    ones = jnp.ones((16,), jnp.float32)
    lane = lax.iota(jnp.int32, 16)
    last = jnp.full((16,), 15, jnp.int32)
    _gdn = lax.GatherDimensionNumbers(
        offset_dims=(), collapsed_slice_dims=(0,), start_index_map=(0,)
    )

    def _permute(v, idx):
        return lax.gather(
            v,
            idx[:, None],
            dimension_numbers=_gdn,
            slice_sizes=(1,),
            mode=lax.GatherScatterMode.PROMISE_IN_BOUNDS,
        )

    def per_batch(b, carry_unused):
        carry = jnp.zeros((16,), jnp.int32)
        for k in range(words // 16):
            d = dur_v[pl.ds(b * words + k * 16, 16)]
            ci = d
            for sh in (1, 2, 4, 8):
                g = _permute(ci, jnp.maximum(lane - sh, 0))
                ci = ci + jnp.where(lane >= sh, g, 0)
            ci = ci + carry
            ce = ci - d
            carry = _permute(ci, last)
            jv = lane + k * 16
            for o in range(15):
                p = ce + o
                valid = (d > o) & (p >= s0) & (p < s0 + s_per_w)
                row = jnp.clip(p - s0, 0, s_per_w - 1)
                idx = (row * batches + b) * 128 + jv
                plsc.store_scatter(oh_v, [idx], ones, mask=valid)
        return carry_unused

    lax.fori_loop(0, batches, per_batch, 0)
    pltpu.sync_copy(oh_v, oh_hbm.at[pl.ds(wid * slab, slab)])


def _tc1_body(dur_ref, pe_ref, x_ref, o_ref, *, sblk, batches, words, blk_off):
    sidx = pl.program_id(0) + blk_off
    rows = sblk * batches

    dur = dur_ref[...].astype(jnp.float32)
    tri = (
        jax.lax.broadcasted_iota(jnp.int32, (words, words), 0)
        <= jax.lax.broadcasted_iota(jnp.int32, (words, words), 1)
    ).astype(jnp.float32)
    csum_in = jnp.dot(dur, tri, preferred_element_type=jnp.float32)
    csum_ex = csum_in - dur

    ci_t = jnp.broadcast_to(csum_in[None], (sblk, batches, words)).reshape(
        rows, words
    )
    ce_t = jnp.broadcast_to(csum_ex[None], (sblk, batches, words)).reshape(
        rows, words
    )
    pos = (
        (jax.lax.broadcasted_iota(jnp.int32, (rows, words), 0) // batches)
        + sidx * sblk
    ).astype(jnp.float32)

    onehot = ((pos >= ce_t) & (pos < ci_t)).astype(jnp.bfloat16)
    add = jnp.dot(
        onehot, pe_ref[...].astype(jnp.bfloat16), preferred_element_type=jnp.float32
    )
    chans = pe_ref.shape[1]
    xb = x_ref[...].reshape(rows, chans)
    o_ref[...] = (xb + add).reshape(sblk, batches, chans)


def _tc2_body(oh_ref, pe_ref, x_ref, prev_ref, o_ref, *, sblk, batches):
    del prev_ref
    rows = sblk * batches
    oh = oh_ref[...].astype(jnp.bfloat16)
    add = jnp.dot(
        oh, pe_ref[...].astype(jnp.bfloat16), preferred_element_type=jnp.float32
    )
    chans = x_ref.shape[2]
    xb = x_ref[...].reshape(rows, chans)
    o_ref[...] = (xb + add).reshape(sblk, batches, chans)


def kernel(x, pe, text_duration, train):
    del train
    S, B, C = x.shape
    _, W = text_duration.shape
    pe_trunc = pe[:W, :]
    dur = text_duration.astype(jnp.int32)
    sblk = 256
    s2 = _NW * 16
    s1 = S - s2
    s_per_w = s2 // _NW
    slab = s_per_w * B * 128

    oh_flat = pl.kernel(
        functools.partial(
            _sc_onehot_body, s_per_w=s_per_w, batches=B, words=W
        ),
        out_type=jax.ShapeDtypeStruct((s2 * B * 128,), jnp.float32),
        mesh=plsc.VectorSubcoreMesh(
            core_axis_name="c",
            subcore_axis_name="s",
            num_cores=_NC,
            num_subcores=_NS,
        ),
        scratch_types=[
            pltpu.VMEM((B * W,), jnp.int32),
            pltpu.VMEM((slab,), jnp.float32),
        ],
        compiler_params=pltpu.CompilerParams(needs_layout_passes=False),
    )(dur.reshape(B * W))
    oh2d = oh_flat.reshape(s2 * B, 128)

    n2 = s2 // sblk
    out1 = pl.pallas_call(
        functools.partial(_tc1_body, sblk=sblk, batches=B, words=W, blk_off=n2),
        grid=(s1 // sblk,),
        in_specs=[
            pl.BlockSpec((B, W), lambda s: (0, 0)),
            pl.BlockSpec((W, C), lambda s: (0, 0)),
            pl.BlockSpec((sblk, B, C), lambda s: (s + n2, 0, 0)),
        ],
        out_specs=pl.BlockSpec((sblk, B, C), lambda s: (s + n2, 0, 0)),
        out_shape=jax.ShapeDtypeStruct((S, B, C), x.dtype),
    )(dur, pe_trunc, x)

    out = pl.pallas_call(
        functools.partial(_tc2_body, sblk=sblk, batches=B),
        grid=(s2 // sblk,),
        in_specs=[
            pl.BlockSpec((sblk * B, 128), lambda s: (s, 0)),
            pl.BlockSpec((W, C), lambda s: (0, 0)),
            pl.BlockSpec((sblk, B, C), lambda s: (s, 0, 0)),
            pl.BlockSpec(memory_space=pltpu.MemorySpace.HBM),
        ],
        out_specs=pl.BlockSpec((sblk, B, C), lambda s: (s, 0, 0)),
        out_shape=jax.ShapeDtypeStruct((S, B, C), x.dtype),
        input_output_aliases={3: 0},
    )(oh2d, pe_trunc, x, out1)
    return out


# speedup vs baseline: 1.0274x; 1.0274x over previous
import functools

import jax
import jax.numpy as jnp
from jax import lax
from jax.experimental import pallas as pl
from jax.experimental.pallas import tpu as pltpu
from jax.experimental.pallas import tpu_sc as plsc

_NC = 2
_NS = 16
_NW = _NC * _NS


def _sc_onehot_body(dur_hbm, oh_hbm, dur_v, oh_v, *, s_win, words):
    wid = lax.axis_index("s") * _NC + lax.axis_index("c")
    b = wid // 2
    h = wid % 2
    pltpu.sync_copy(dur_hbm.at[pl.ds(b * words, words)], dur_v)

    zeros16 = jnp.zeros((16,), jnp.float32)

    def zero_fill(i, carry):
        for u in range(8):
            oh_v[i, pl.ds(u * 16, 16)] = zeros16
        return carry

    lax.fori_loop(0, s_win, zero_fill, 0)

    p0 = h * s_win
    ones = jnp.ones((16,), jnp.float32)
    lane = lax.iota(jnp.int32, 16)
    last = jnp.full((16,), 15, jnp.int32)
    _gdn = lax.GatherDimensionNumbers(
        offset_dims=(), collapsed_slice_dims=(0,), start_index_map=(0,)
    )

    def _permute(v, idx):
        return lax.gather(
            v,
            idx[:, None],
            dimension_numbers=_gdn,
            slice_sizes=(1,),
            mode=lax.GatherScatterMode.PROMISE_IN_BOUNDS,
        )

    carry = jnp.zeros((16,), jnp.int32)
    for k in range(words // 16):
        d = dur_v[pl.ds(k * 16, 16)]
        ci = d
        for sh in (1, 2, 4, 8):
            g = _permute(ci, jnp.maximum(lane - sh, 0))
            ci = ci + jnp.where(lane >= sh, g, 0)
        ci = ci + carry
        ce = ci - d
        carry = _permute(ci, last)
        jv = lane + k * 16
        for o in range(15):
            p = ce + o
            valid = (d > o) & (p >= p0) & (p < p0 + s_win)
            row = jnp.clip(p - p0, 0, s_win - 1)
            plsc.store_scatter(oh_v, [row, jv], ones, mask=valid)

    pltpu.sync_copy(oh_v, oh_hbm.at[pl.ds(p0, s_win), b])


def _tc1_body(dur_ref, pe_ref, x_ref, o_ref, *, sblk, batches, words, blk_off):
    sidx = pl.program_id(0) + blk_off
    rows = sblk * batches

    dur = dur_ref[...].astype(jnp.float32)
    tri = (
        jax.lax.broadcasted_iota(jnp.int32, (words, words), 0)
        <= jax.lax.broadcasted_iota(jnp.int32, (words, words), 1)
    ).astype(jnp.float32)
    csum_in = jnp.dot(dur, tri, preferred_element_type=jnp.float32)
    csum_ex = csum_in - dur

    ci_t = jnp.broadcast_to(csum_in[None], (sblk, batches, words)).reshape(
        rows, words
    )
    ce_t = jnp.broadcast_to(csum_ex[None], (sblk, batches, words)).reshape(
        rows, words
    )
    pos = (
        (jax.lax.broadcasted_iota(jnp.int32, (rows, words), 0) // batches)
        + sidx * sblk
    ).astype(jnp.float32)

    onehot = ((pos >= ce_t) & (pos < ci_t)).astype(jnp.bfloat16)
    add = jnp.dot(
        onehot, pe_ref[...].astype(jnp.bfloat16), preferred_element_type=jnp.float32
    )
    chans = pe_ref.shape[1]
    xb = x_ref[...].reshape(rows, chans)
    o_ref[...] = (xb + add).reshape(sblk, batches, chans)


def _tc2_body(oh_ref, pe_ref, x_ref, prev_ref, o_ref, *, sblk, batches):
    del prev_ref
    rows = sblk * batches
    oh = oh_ref[...].astype(jnp.bfloat16)
    add = jnp.dot(
        oh, pe_ref[...].astype(jnp.bfloat16), preferred_element_type=jnp.float32
    )
    chans = x_ref.shape[2]
    xb = x_ref[...].reshape(rows, chans)
    o_ref[...] = (xb + add).reshape(sblk, batches, chans)


def kernel(x, pe, text_duration, train):
    del train
    S, B, C = x.shape
    _, W = text_duration.shape
    pe_trunc = pe[:W, :]
    dur = text_duration.astype(jnp.int32)
    sblk = 256
    s2 = 512
    s1 = S - s2
    s_win = s2 * B // _NW

    oh3d = pl.kernel(
        functools.partial(_sc_onehot_body, s_win=s_win, words=W),
        out_type=jax.ShapeDtypeStruct((s2, B, 128), jnp.float32),
        mesh=plsc.VectorSubcoreMesh(
            core_axis_name="c",
            subcore_axis_name="s",
            num_cores=_NC,
            num_subcores=_NS,
        ),
        scratch_types=[
            pltpu.VMEM((W,), jnp.int32),
            pltpu.VMEM((s_win, 128), jnp.float32),
        ],
        compiler_params=pltpu.CompilerParams(needs_layout_passes=False),
    )(dur.reshape(B * W))
    oh2d = oh3d.reshape(s2 * B, 128)

    n2 = s2 // sblk
    out1 = pl.pallas_call(
        functools.partial(_tc1_body, sblk=sblk, batches=B, words=W, blk_off=n2),
        grid=(s1 // sblk,),
        in_specs=[
            pl.BlockSpec((B, W), lambda s: (0, 0)),
            pl.BlockSpec((W, C), lambda s: (0, 0)),
            pl.BlockSpec((sblk, B, C), lambda s: (s + n2, 0, 0)),
        ],
        out_specs=pl.BlockSpec((sblk, B, C), lambda s: (s + n2, 0, 0)),
        out_shape=jax.ShapeDtypeStruct((S, B, C), x.dtype),
    )(dur, pe_trunc, x)

    out = pl.pallas_call(
        functools.partial(_tc2_body, sblk=sblk, batches=B),
        grid=(s2 // sblk,),
        in_specs=[
            pl.BlockSpec((sblk * B, 128), lambda s: (s, 0)),
            pl.BlockSpec((W, C), lambda s: (0, 0)),
            pl.BlockSpec((sblk, B, C), lambda s: (s, 0, 0)),
            pl.BlockSpec(memory_space=pltpu.MemorySpace.HBM),
        ],
        out_specs=pl.BlockSpec((sblk, B, C), lambda s: (s, 0, 0)),
        out_shape=jax.ShapeDtypeStruct((S, B, C), x.dtype),
        input_output_aliases={3: 0},
    )(oh2d, pe_trunc, x, out1)
    return out


# speedup vs baseline: 1.0308x; 1.0033x over previous
import functools

import jax
import jax.numpy as jnp
from jax import lax
from jax.experimental import pallas as pl
from jax.experimental.pallas import tpu as pltpu
from jax.experimental.pallas import tpu_sc as plsc

_NC = 2
_NS = 16
_NW = _NC * _NS


def _sc_onehot_body(dur_hbm, oh_hbm, dur_v, oh_v, *, s_win, words):
    wid = lax.axis_index("s") * _NC + lax.axis_index("c")
    b = wid // 2
    h = wid % 2
    pltpu.sync_copy(dur_hbm.at[pl.ds(b * words, words)], dur_v)

    zeros16 = jnp.zeros((16,), jnp.float32)

    def zero_fill(i, carry):
        for u in range(8):
            oh_v[i, pl.ds(u * 16, 16)] = zeros16
        return carry

    lax.fori_loop(0, s_win, zero_fill, 0)

    p0 = h * s_win
    ones = jnp.ones((16,), jnp.float32)
    lane = lax.iota(jnp.int32, 16)
    last = jnp.full((16,), 15, jnp.int32)
    _gdn = lax.GatherDimensionNumbers(
        offset_dims=(), collapsed_slice_dims=(0,), start_index_map=(0,)
    )

    def _permute(v, idx):
        return lax.gather(
            v,
            idx[:, None],
            dimension_numbers=_gdn,
            slice_sizes=(1,),
            mode=lax.GatherScatterMode.PROMISE_IN_BOUNDS,
        )

    carry = jnp.zeros((16,), jnp.int32)
    for k in range(words // 16):
        d = dur_v[pl.ds(k * 16, 16)]
        ci = d
        for sh in (1, 2, 4, 8):
            g = _permute(ci, jnp.maximum(lane - sh, 0))
            ci = ci + jnp.where(lane >= sh, g, 0)
        ci = ci + carry
        ce = ci - d
        carry = _permute(ci, last)
        jv = lane + k * 16
        for o in range(15):
            p = ce + o
            valid = (d > o) & (p >= p0) & (p < p0 + s_win)
            row = jnp.clip(p - p0, 0, s_win - 1)
            plsc.store_scatter(oh_v, [row, jv], ones, mask=valid)

    pltpu.sync_copy(oh_v, oh_hbm.at[pl.ds(p0, s_win), b])


def _tc1_body(dur_ref, pe_ref, x_ref, o_ref, *, sblk, batches, words, blk_off):
    sidx = pl.program_id(0) + blk_off
    rows = sblk * batches

    dur = dur_ref[...].astype(jnp.float32)
    tri = (
        jax.lax.broadcasted_iota(jnp.int32, (words, words), 0)
        <= jax.lax.broadcasted_iota(jnp.int32, (words, words), 1)
    ).astype(jnp.float32)
    csum_in = jnp.dot(dur, tri, preferred_element_type=jnp.float32)
    csum_ex = csum_in - dur

    ci_t = jnp.broadcast_to(csum_in[None], (sblk, batches, words)).reshape(
        rows, words
    )
    ce_t = jnp.broadcast_to(csum_ex[None], (sblk, batches, words)).reshape(
        rows, words
    )
    pos = (
        (jax.lax.broadcasted_iota(jnp.int32, (rows, words), 0) // batches)
        + sidx * sblk
    ).astype(jnp.float32)

    onehot = ((pos >= ce_t) & (pos < ci_t)).astype(jnp.bfloat16)
    add = jnp.dot(
        onehot, pe_ref[...].astype(jnp.bfloat16), preferred_element_type=jnp.float32
    )
    chans = pe_ref.shape[1]
    xb = x_ref[...].reshape(rows, chans)
    o_ref[...] = (xb + add).reshape(sblk, batches, chans)


def _tc2_body(oh_ref, pe_ref, x_ref, prev_ref, o_ref, *, sblk, batches):
    del prev_ref
    rows = sblk * batches
    oh = oh_ref[...].astype(jnp.bfloat16)
    add = jnp.dot(
        oh, pe_ref[...].astype(jnp.bfloat16), preferred_element_type=jnp.float32
    )
    chans = x_ref.shape[2]
    xb = x_ref[...].reshape(rows, chans)
    o_ref[...] = (xb + add).reshape(sblk, batches, chans)


def kernel(x, pe, text_duration, train):
    del train
    S, B, C = x.shape
    _, W = text_duration.shape
    pe_trunc = pe[:W, :]
    dur = text_duration.astype(jnp.int32)
    sblk = 256
    s2 = 512
    s1 = S - s2
    s_win = s2 * B // _NW

    oh3d = pl.kernel(
        functools.partial(_sc_onehot_body, s_win=s_win, words=W),
        out_type=jax.ShapeDtypeStruct((s2, B, 128), jnp.float32),
        mesh=plsc.VectorSubcoreMesh(
            core_axis_name="c",
            subcore_axis_name="s",
            num_cores=_NC,
            num_subcores=_NS,
        ),
        scratch_types=[
            pltpu.VMEM((W,), jnp.int32),
            pltpu.VMEM((s_win, 128), jnp.float32),
        ],
        compiler_params=pltpu.CompilerParams(needs_layout_passes=False),
    )(dur.reshape(B * W))
    oh2d = oh3d.reshape(s2 * B, 128)

    n2 = s2 // sblk
    out1 = pl.pallas_call(
        functools.partial(_tc1_body, sblk=sblk, batches=B, words=W, blk_off=n2),
        grid=(s1 // sblk,),
        in_specs=[
            pl.BlockSpec((B, W), lambda s: (0, 0)),
            pl.BlockSpec((W, C), lambda s: (0, 0)),
            pl.BlockSpec((sblk, B, C), lambda s: (s + n2, 0, 0)),
        ],
        out_specs=pl.BlockSpec((sblk, B, C), lambda s: (s + n2, 0, 0)),
        out_shape=jax.ShapeDtypeStruct((S, B, C), x.dtype),
        compiler_params=pltpu.CompilerParams(skip_device_barrier=True),
    )(dur, pe_trunc, x)

    out = pl.pallas_call(
        functools.partial(_tc2_body, sblk=sblk, batches=B),
        grid=(s2 // sblk,),
        in_specs=[
            pl.BlockSpec((sblk * B, 128), lambda s: (s, 0)),
            pl.BlockSpec((W, C), lambda s: (0, 0)),
            pl.BlockSpec((sblk, B, C), lambda s: (s, 0, 0)),
            pl.BlockSpec(memory_space=pltpu.MemorySpace.HBM),
        ],
        out_specs=pl.BlockSpec((sblk, B, C), lambda s: (s, 0, 0)),
        out_shape=jax.ShapeDtypeStruct((S, B, C), x.dtype),
        input_output_aliases={3: 0},
    )(oh2d, pe_trunc, x, out1)
    return out


# speedup vs baseline: 1.0501x; 1.0186x over previous
import functools

import jax
import jax.numpy as jnp
from jax import lax
from jax.experimental import pallas as pl
from jax.experimental.pallas import tpu as pltpu
from jax.experimental.pallas import tpu_sc as plsc

_NC = 2
_NS = 16
_NW = _NC * _NS


def _sc_onehot_body(dur_hbm, oh_hbm, dur_v, oh_v, *, s_win, words):
    wid = lax.axis_index("s") * _NC + lax.axis_index("c")
    b = wid // 2
    h = wid % 2
    pltpu.sync_copy(dur_hbm.at[pl.ds(b * words, words)], dur_v)

    zeros16 = jnp.zeros((16,), jnp.float32)

    def zero_fill(i, carry):
        for u in range(8):
            oh_v[i, pl.ds(u * 16, 16)] = zeros16
        return carry

    lax.fori_loop(0, s_win, zero_fill, 0)

    p0 = h * s_win
    ones = jnp.ones((16,), jnp.float32)
    lane = lax.iota(jnp.int32, 16)
    last = jnp.full((16,), 15, jnp.int32)
    _gdn = lax.GatherDimensionNumbers(
        offset_dims=(), collapsed_slice_dims=(0,), start_index_map=(0,)
    )

    def _permute(v, idx):
        return lax.gather(
            v,
            idx[:, None],
            dimension_numbers=_gdn,
            slice_sizes=(1,),
            mode=lax.GatherScatterMode.PROMISE_IN_BOUNDS,
        )

    carry = jnp.zeros((16,), jnp.int32)
    for k in range(words // 16):
        d = dur_v[pl.ds(k * 16, 16)]
        ci = d
        for sh in (1, 2, 4, 8):
            g = _permute(ci, jnp.maximum(lane - sh, 0))
            ci = ci + jnp.where(lane >= sh, g, 0)
        ci = ci + carry
        ce = ci - d
        carry = _permute(ci, last)
        jv = lane + k * 16
        for o in range(15):
            p = ce + o
            valid = (d > o) & (p >= p0) & (p < p0 + s_win)
            row = jnp.clip(p - p0, 0, s_win - 1)
            plsc.store_scatter(oh_v, [row, jv], ones, mask=valid)

    pltpu.sync_copy(oh_v, oh_hbm.at[pl.ds(p0, s_win), b])


def _tc1_body(dur_ref, pe_ref, x_ref, o_ref, *, sblk, batches, words, blk_off):
    sidx = pl.program_id(0) + blk_off
    rows = sblk * batches

    dur = dur_ref[...].astype(jnp.float32)
    tri = (
        jax.lax.broadcasted_iota(jnp.int32, (words, words), 0)
        <= jax.lax.broadcasted_iota(jnp.int32, (words, words), 1)
    ).astype(jnp.float32)
    csum_in = jnp.dot(dur, tri, preferred_element_type=jnp.float32)
    csum_ex = csum_in - dur

    ci_t = jnp.broadcast_to(csum_in[None], (sblk, batches, words)).reshape(
        rows, words
    )
    ce_t = jnp.broadcast_to(csum_ex[None], (sblk, batches, words)).reshape(
        rows, words
    )
    pos = (
        (jax.lax.broadcasted_iota(jnp.int32, (rows, words), 0) // batches)
        + sidx * sblk
    ).astype(jnp.float32)

    onehot = ((pos >= ce_t) & (pos < ci_t)).astype(jnp.bfloat16)
    add = jnp.dot(
        onehot, pe_ref[...].astype(jnp.bfloat16), preferred_element_type=jnp.float32
    )
    chans = pe_ref.shape[1]
    xb = x_ref[...].reshape(rows, chans)
    o_ref[...] = (xb + add).reshape(sblk, batches, chans)


def _tc2_body(oh_ref, pe_ref, x_ref, prev_ref, o_ref, *, sblk, batches):
    del prev_ref
    rows = sblk * batches
    oh = oh_ref[...].astype(jnp.bfloat16)
    add = jnp.dot(
        oh, pe_ref[...].astype(jnp.bfloat16), preferred_element_type=jnp.float32
    )
    chans = x_ref.shape[2]
    xb = x_ref[...].reshape(rows, chans)
    o_ref[...] = (xb + add).reshape(sblk, batches, chans)


def kernel(x, pe, text_duration, train):
    del train
    S, B, C = x.shape
    _, W = text_duration.shape
    pe_trunc = pe[:W, :]
    dur = text_duration.astype(jnp.int32)
    sblk = 256
    s2 = 256
    s1 = S - s2
    s_win = s2 * B // _NW

    oh3d = pl.kernel(
        functools.partial(_sc_onehot_body, s_win=s_win, words=W),
        out_type=jax.ShapeDtypeStruct((s2, B, 128), jnp.float32),
        mesh=plsc.VectorSubcoreMesh(
            core_axis_name="c",
            subcore_axis_name="s",
            num_cores=_NC,
            num_subcores=_NS,
        ),
        scratch_types=[
            pltpu.VMEM((W,), jnp.int32),
            pltpu.VMEM((s_win, 128), jnp.float32),
        ],
        compiler_params=pltpu.CompilerParams(needs_layout_passes=False),
    )(dur.reshape(B * W))
    oh2d = oh3d.reshape(s2 * B, 128)

    n2 = s2 // sblk
    out1 = pl.pallas_call(
        functools.partial(_tc1_body, sblk=sblk, batches=B, words=W, blk_off=n2),
        grid=(s1 // sblk,),
        in_specs=[
            pl.BlockSpec((B, W), lambda s: (0, 0)),
            pl.BlockSpec((W, C), lambda s: (0, 0)),
            pl.BlockSpec((sblk, B, C), lambda s: (s + n2, 0, 0)),
        ],
        out_specs=pl.BlockSpec((sblk, B, C), lambda s: (s + n2, 0, 0)),
        out_shape=jax.ShapeDtypeStruct((S, B, C), x.dtype),
    )(dur, pe_trunc, x)

    out = pl.pallas_call(
        functools.partial(_tc2_body, sblk=sblk, batches=B),
        grid=(s2 // sblk,),
        in_specs=[
            pl.BlockSpec((sblk * B, 128), lambda s: (s, 0)),
            pl.BlockSpec((W, C), lambda s: (0, 0)),
            pl.BlockSpec((sblk, B, C), lambda s: (s, 0, 0)),
            pl.BlockSpec(memory_space=pltpu.MemorySpace.HBM),
        ],
        out_specs=pl.BlockSpec((sblk, B, C), lambda s: (s, 0, 0)),
        out_shape=jax.ShapeDtypeStruct((S, B, C), x.dtype),
        input_output_aliases={3: 0},
    )(oh2d, pe_trunc, x, out1)
    return out
